# baseline (speedup 1.0000x reference)
import functools

import jax
import jax.numpy as jnp
from jax import lax
from jax.experimental import pallas as pl
from jax.experimental.pallas import tpu as pltpu
from jax.experimental.pallas import tpu_sc as plsc

_R = 5
_N = 4096
_L = 4096
_NC = 2
_NS = 16
_NW = _NC * _NS
_ROWS = _N // _NW
_LANES = 16

_NEGINF = float("-inf")
_POSINF = float("inf")


def _perm(v, idx):
    return lax.gather(
        v, idx[:, None],
        lax.GatherDimensionNumbers(offset_dims=(), collapsed_slice_dims=(0,),
                                   start_index_map=(0,)),
        slice_sizes=(1,), mode=lax.GatherScatterMode.PROMISE_IN_BOUNDS)


def _row_result(buf, leff):
    iota = lax.iota(jnp.int32, _LANES)
    vec = jnp.full((_LANES,), _POSINF, jnp.float32)
    for j in range(_R):
        s = (j * leff) // _R
        e = ((j + 1) * leff + (_R - 1)) // _R
        c0 = (s // _LANES) * _LANES
        v0 = buf[pl.ds(pl.multiple_of(c0, _LANES), _LANES)]
        m0 = (iota >= s - c0) & (iota < e - c0)
        am0 = jnp.where(m0, v0, _NEGINF)
        an0 = jnp.where(m0, v0, _POSINF)
        n_int = jnp.maximum((e - c0) // _LANES - 1, 0)
        c1 = c0 + _LANES

        @plsc.parallel_loop(0, n_int, unroll=8, carry=(am0, an0))
        def _interior(t, carry, c1=c1):
            am_, an_ = carry
            v = buf[pl.ds(pl.multiple_of(c1 + t * _LANES, _LANES), _LANES)]
            return jnp.maximum(am_, v), jnp.minimum(an_, v)

        am, an = _interior
        pt = c1 + n_int * _LANES
        vt = buf[pl.ds(pl.multiple_of(jnp.minimum(pt, _L - _LANES), _LANES),
                       _LANES)]
        mt = iota < (e - pt)
        am = jnp.maximum(am, jnp.where(mt, vt, _NEGINF))
        an = jnp.minimum(an, jnp.where(mt, vt, _POSINF))
        for sh in (1, 2, 4, 8):
            am = jnp.maximum(am, _perm(am, iota ^ sh))
            an = jnp.minimum(an, _perm(an, iota ^ sh))
        vec = jnp.where(iota == j, an, vec)
        vec = jnp.where(iota == (_R + j), am, vec)
    for k in (2, 4, 8, 16):
        sh = k // 2
        while sh >= 1:
            p = _perm(vec, iota ^ sh)
            want_min = ((iota & sh) == 0) != ((iota & k) != 0)
            vec = jnp.where(want_min, jnp.minimum(vec, p),
                            jnp.maximum(vec, p))
            sh //= 2
    return vec


def _sc_body(x_hbm, len_hbm, out_hbm, len_v, buf0, buf1, outv, sem0, sem1):
    wid = lax.axis_index("s") * _NC + lax.axis_index("c")
    base = wid * _ROWS
    pltpu.sync_copy(len_hbm.at[pl.ds(base * _LANES, _ROWS * _LANES)], len_v)
    bufs = (buf0, buf1)
    sems = (sem0, sem1)
    pltpu.async_copy(x_hbm.at[base], buf0, sem0)

    def outer(i2, _):
        for k in range(2):
            i = i2 * 2 + k
            cur = bufs[k]
            nxt = bufs[1 - k]

            @pl.when(i + 1 < _ROWS)
            def _():
                pltpu.async_copy(x_hbm.at[base + i + 1], nxt, sems[1 - k])

            pltpu.make_async_copy(x_hbm.at[base + i], cur, sems[k]).wait()
            lv = len_v[pl.ds(pl.multiple_of(i * _LANES, _LANES), _LANES)]
            leff = lv[0]
            outv[i, :] = _row_result(cur, leff)
        return 0

    lax.fori_loop(0, _ROWS // 2, outer, 0)
    pltpu.sync_copy(outv, out_hbm.at[pl.ds(base, _ROWS)])


@jax.jit
def _minmax16(inputs, lengths16):
    mesh = plsc.VectorSubcoreMesh(core_axis_name="c", subcore_axis_name="s")
    f = functools.partial(
        pl.kernel,
        out_type=jax.ShapeDtypeStruct((_N, _LANES), jnp.float32),
        mesh=mesh,
        scratch_types=[
            pltpu.VMEM((_ROWS * _LANES,), jnp.int32),
            pltpu.VMEM((_L,), jnp.float32),
            pltpu.VMEM((_L,), jnp.float32),
            pltpu.VMEM((_ROWS, _LANES), jnp.float32),
            pltpu.SemaphoreType.DMA,
            pltpu.SemaphoreType.DMA,
        ],
    )(_sc_body)
    return f(inputs, lengths16)


def kernel(inputs, lengths):
    lengths16 = jnp.repeat(jnp.clip(lengths.astype(jnp.int32), 1, _L), _LANES)
    out16 = _minmax16(inputs, lengths16)
    return out16[:, : 2 * _R]


# speedup vs baseline: 1.8057x; 1.8057x over previous
import functools

import jax
import jax.numpy as jnp
from jax import lax
from jax.experimental import pallas as pl
from jax.experimental.pallas import tpu as pltpu
from jax.experimental.pallas import tpu_sc as plsc

_R = 5
_N = 4096
_L = 4096
_NC = 2
_NS = 16
_NW = _NC * _NS
_ROWS = _N // _NW
_LANES = 16

_NEGINF = float("-inf")
_POSINF = float("inf")


def _perm(v, idx):
    return lax.gather(
        v, idx[:, None],
        lax.GatherDimensionNumbers(offset_dims=(), collapsed_slice_dims=(0,),
                                   start_index_map=(0,)),
        slice_sizes=(1,), mode=lax.GatherScatterMode.PROMISE_IN_BOUNDS)


def _row_result(buf, leff):
    iota = lax.iota(jnp.int32, _LANES)
    vec = jnp.full((_LANES,), _POSINF, jnp.float32)
    for j in range(_R):
        s = (j * leff) // _R
        e = ((j + 1) * leff + (_R - 1)) // _R
        c0 = (s // _LANES) * _LANES
        v0 = buf[pl.ds(pl.multiple_of(c0, _LANES), _LANES)]
        m0 = (iota >= s - c0) & (iota < e - c0)
        am0 = jnp.where(m0, v0, _NEGINF)
        an0 = jnp.where(m0, v0, _POSINF)
        n_int = jnp.maximum((e - c0) // _LANES - 1, 0)
        c1 = c0 + _LANES
        n_grp = n_int // 8

        def _group(base, am_, an_):
            vs = [buf[pl.ds(pl.multiple_of(base + u * _LANES, _LANES),
                            _LANES)] for u in range(8)]
            mx = jnp.maximum(jnp.maximum(jnp.maximum(vs[0], vs[1]),
                                         jnp.maximum(vs[2], vs[3])),
                             jnp.maximum(jnp.maximum(vs[4], vs[5]),
                                         jnp.maximum(vs[6], vs[7])))
            mn = jnp.minimum(jnp.minimum(jnp.minimum(vs[0], vs[1]),
                                         jnp.minimum(vs[2], vs[3])),
                             jnp.minimum(jnp.minimum(vs[4], vs[5]),
                                         jnp.minimum(vs[6], vs[7])))
            return jnp.maximum(am_, mx), jnp.minimum(an_, mn)

        @plsc.parallel_loop(0, n_grp, carry=(am0, an0))
        def _interior(g, carry, c1=c1):
            return _group(c1 + g * (8 * _LANES), *carry)

        am, an = _interior
        base_o = c1 + jnp.maximum(n_int - 8, 0) * _LANES
        am_o, an_o = _group(base_o, am, an)
        big = n_int >= 8
        am = jnp.where(big, am_o, am)
        an = jnp.where(big, an_o, an)

        @plsc.parallel_loop(0, jnp.where(big, 0, n_int), carry=(am, an))
        def _singles(t, carry, c1=c1):
            am_, an_ = carry
            v = buf[pl.ds(pl.multiple_of(c1 + t * _LANES, _LANES), _LANES)]
            return jnp.maximum(am_, v), jnp.minimum(an_, v)

        am, an = _singles
        pt = c1 + n_int * _LANES
        vt = buf[pl.ds(pl.multiple_of(jnp.minimum(pt, _L - _LANES), _LANES),
                       _LANES)]
        mt = iota < (e - pt)
        am = jnp.maximum(am, jnp.where(mt, vt, _NEGINF))
        an = jnp.minimum(an, jnp.where(mt, vt, _POSINF))
        for sh in (1, 2, 4, 8):
            am = jnp.maximum(am, _perm(am, iota ^ sh))
            an = jnp.minimum(an, _perm(an, iota ^ sh))
        vec = jnp.where(iota == j, an, vec)
        vec = jnp.where(iota == (_R + j), am, vec)
    for k in (2, 4, 8, 16):
        sh = k // 2
        while sh >= 1:
            p = _perm(vec, iota ^ sh)
            want_min = ((iota & sh) == 0) != ((iota & k) != 0)
            vec = jnp.where(want_min, jnp.minimum(vec, p),
                            jnp.maximum(vec, p))
            sh //= 2
    return vec


def _sc_body(x_hbm, len_hbm, out_hbm, len_v, buf0, buf1, outv, sem0, sem1):
    wid = lax.axis_index("s") * _NC + lax.axis_index("c")
    base = wid * _ROWS
    pltpu.sync_copy(len_hbm.at[pl.ds(base * _LANES, _ROWS * _LANES)], len_v)
    bufs = (buf0, buf1)
    sems = (sem0, sem1)
    pltpu.async_copy(x_hbm.at[base], buf0, sem0)

    def outer(i2, _):
        for k in range(2):
            i = i2 * 2 + k
            cur = bufs[k]
            nxt = bufs[1 - k]

            @pl.when(i + 1 < _ROWS)
            def _():
                pltpu.async_copy(x_hbm.at[base + i + 1], nxt, sems[1 - k])

            pltpu.make_async_copy(x_hbm.at[base + i], cur, sems[k]).wait()
            lv = len_v[pl.ds(pl.multiple_of(i * _LANES, _LANES), _LANES)]
            leff = lv[0]
            outv[i, :] = _row_result(cur, leff)
        return 0

    lax.fori_loop(0, _ROWS // 2, outer, 0)
    pltpu.sync_copy(outv, out_hbm.at[pl.ds(base, _ROWS)])


@jax.jit
def _minmax16(inputs, lengths16):
    mesh = plsc.VectorSubcoreMesh(core_axis_name="c", subcore_axis_name="s")
    f = functools.partial(
        pl.kernel,
        out_type=jax.ShapeDtypeStruct((_N, _LANES), jnp.float32),
        mesh=mesh,
        scratch_types=[
            pltpu.VMEM((_ROWS * _LANES,), jnp.int32),
            pltpu.VMEM((_L,), jnp.float32),
            pltpu.VMEM((_L,), jnp.float32),
            pltpu.VMEM((_ROWS, _LANES), jnp.float32),
            pltpu.SemaphoreType.DMA,
            pltpu.SemaphoreType.DMA,
        ],
    )(_sc_body)
    return f(inputs, lengths16)


def kernel(inputs, lengths):
    lengths16 = jnp.repeat(jnp.clip(lengths.astype(jnp.int32), 1, _L), _LANES)
    out16 = _minmax16(inputs, lengths16)
    return out16[:, : 2 * _R]
